# baseline (speedup 1.0000x reference)
import functools

import jax
import jax.numpy as jnp
from jax import lax
from jax.experimental import pallas as pl
from jax.experimental.pallas import tpu as pltpu
from jax.experimental.pallas import tpu_sc as plsc

_info = plsc.get_sparse_core_info()
_NC, _NS = _info.num_cores, _info.num_subcores
_NW = _NC * _NS

_V = 1000001
_B = 16384
_D = 64
_STRIPE = 31360
_W = 512
_NSLAB = -(-_STRIPE // _W)
_VPAD = 1000064
_DUMP = _B
_LCAP = _B + 64
_LDUMP = _B + 16


def _make_emb():
    mesh = plsc.VectorSubcoreMesh(core_axis_name="c", subcore_axis_name="s")

    @functools.partial(
        pl.kernel,
        mesh=mesh,
        out_type=jax.ShapeDtypeStruct((_B + 8, 128), jnp.float32),
        scratch_types=[
            pltpu.VMEM((_B,), jnp.int32),
            pltpu.VMEM((_LCAP,), jnp.int32),
            pltpu.VMEM((_LCAP,), jnp.int32),
            pltpu.VMEM((_D, _W), jnp.float32),
            pltpu.VMEM((16, 128), jnp.float32),
            pltpu.SemaphoreType.DMA,
        ],
        compiler_params=pltpu.CompilerParams(needs_layout_passes=False),
    )
    def emb(idx_hbm, tt_hbm, out_hbm, xv, glist, slist, slab, rowbuf, sem):
        wid = lax.axis_index("s") * _NC + lax.axis_index("c")
        lo = wid * _STRIPE
        iota = lax.iota(jnp.int32, 16)
        pltpu.sync_copy(idx_hbm, xv)

        def scan_step(g, cnt):
            for u in range(4):
                gg = g * 4 + u
                v = xv[pl.ds(gg * 16, 16)]
                rel = v - lo
                m = (rel >= 0) & (rel < _STRIPE)
                pref = plsc.cumsum(jnp.where(m, 1, 0))
                pos = jnp.where(m, cnt + pref - 1, _LDUMP + iota)
                packed = (rel << 15) | (iota + gg * 16)
                plsc.store_scatter(glist, [pos], packed)
                cnt = cnt + pref[15]
            return cnt

        cnt = lax.fori_loop(0, _B // 64, scan_step, 0)
        glist[pl.ds(cnt, 16)] = jnp.full(
            (16,), ((_STRIPE + _W) << 15) | _DUMP, jnp.int32)

        def slab_step(s, carry):
            slab_lo = s * _W
            fs = jnp.minimum(lo + slab_lo, _VPAD - _W)
            fs = pl.multiple_of(fs, 128)
            pltpu.sync_copy(tt_hbm.at[:, pl.ds(fs, _W)], slab)
            frel = fs - lo
            p_lo = slab_lo << 15
            p_hi = (slab_lo + _W) << 15

            def rescan_step(q, c2):
                v = glist[pl.ds(q * 16, 16)]
                m = (v >= p_lo) & (v < p_hi)
                pref = plsc.cumsum(jnp.where(m, 1, 0))
                pos = jnp.where(m, c2 + pref - 1, _LDUMP + iota)
                plsc.store_scatter(slist, [pos], v)
                return c2 + pref[15]

            c2 = lax.fori_loop(0, (cnt + 16) // 16, rescan_step, 0)

            @pl.when(c2 > 0)
            def _():
                slist[pl.ds(c2, 16)] = jnp.full(
                    (16,), (frel << 15) | _DUMP, jnp.int32)

                def extract(q, carry2):
                    v = slist[pl.ds(q * 16, 16)]
                    rr = (v >> 15) - frel
                    ii = v & 0x7FFF
                    for j in range(16):
                        rj = jnp.full((16,), rr[j], jnp.int32)
                        for cg in range(4):
                            vals = plsc.load_gather(
                                slab, [iota + cg * 16, rj])
                            rowbuf[j, pl.ds(cg * 16, 16)] = vals
                    pltpu.async_copy(rowbuf, out_hbm.at[ii], sem).wait()
                    return carry2

                lax.fori_loop(0, (c2 + 15) // 16, extract, 0)

            return carry

        lax.fori_loop(0, _NSLAB, slab_step, 0)

    return emb


_emb = _make_emb()


def kernel(x, table):
    out128 = _emb(x.astype(jnp.int32), table.T)
    return out128[:_B, :_D]
